# baseline (speedup 1.0000x reference)
import functools

import jax
import jax.numpy as jnp
from jax import lax
from jax.experimental import pallas as pl
from jax.experimental.pallas import tpu as pltpu
from jax.experimental.pallas import tpu_sc as plsc

B, At, Nbr, D = 2, 512, 24, 64
NC, NS = 2, 16
SLABS = Nbr * Nbr
SPT = SLABS // NS
NBF = At * Nbr
LANES = 16
HALF = D // 2


def _mesh():
    return plsc.VectorSubcoreMesh(
        core_axis_name="c", subcore_axis_name="s", num_cores=NC, num_subcores=NS
    )


@functools.partial(
    pl.kernel,
    out_type=jax.ShapeDtypeStruct((B, Nbr, Nbr, D, At), jnp.float32),
    mesh=_mesh(),
    scratch_types=[
        pltpu.VMEM((At, D), jnp.float32),
        pltpu.VMEM((NBF // 128, 128), jnp.int32),
        pltpu.VMEM((At,), jnp.int32),
        pltpu.VMEM((HALF, At), jnp.float32),
        pltpu.VMEM((HALF, At), jnp.float32),
        pltpu.SemaphoreType.DMA,
        pltpu.SemaphoreType.DMA,
    ],
    compiler_params=pltpu.CompilerParams(needs_layout_passes=False),
)
def _node_k_t(emb, nbr, out, emb_v, nbr_v, t2_ref, buf0, buf1, sem0, sem1):
    c = lax.axis_index("c")
    s = lax.axis_index("s")
    pltpu.sync_copy(emb.at[c], emb_v)
    pltpu.sync_copy(nbr.at[c], nbr_v)
    m0 = s * SPT

    def drain(buf, sem):
        pltpu.make_async_copy(out.at[0, 0, 0, pl.ds(0, HALF)], buf, sem).wait()

    def fill(buf, base_dd):
        def body(ablk, carry):
            t2v = t2_ref[pl.ds(ablk * LANES, LANES)]
            for dl in range(HALF):
                col = jnp.full((LANES,), base_dd + dl, jnp.int32)
                buf[dl, pl.ds(ablk * LANES, LANES)] = plsc.load_gather(
                    emb_v, [t2v, col]
                )
            return carry

        lax.fori_loop(0, At // LANES, body, 0)

    def step(g, carry):
        m = m0 + g
        j = m // Nbr
        k = m - j * Nbr

        def build(ablk, carry2):
            a = ablk * LANES + lax.iota(jnp.int32, LANES)
            f1 = a * Nbr + j
            t1 = plsc.load_gather(nbr_v, [f1 // 128, f1 & 127])
            f2 = t1 * Nbr + k
            t2 = plsc.load_gather(nbr_v, [f2 // 128, f2 & 127])
            t2_ref[pl.ds(ablk * LANES, LANES)] = t2
            return carry2

        lax.fori_loop(0, At // LANES, build, 0)

        @pl.when(g > 0)
        def _():
            drain(buf0, sem0)

        fill(buf0, 0)
        pltpu.async_copy(buf0, out.at[c, j, k, pl.ds(0, HALF)], sem0)

        @pl.when(g > 0)
        def _():
            drain(buf1, sem1)

        fill(buf1, HALF)
        pltpu.async_copy(buf1, out.at[c, j, k, pl.ds(HALF, HALF)], sem1)
        return carry

    lax.fori_loop(0, SPT, step, 0)
    drain(buf0, sem0)
    drain(buf1, sem1)


def kernel(node_embedding, nbr_idx):
    b, at, d = node_embedding.shape
    nbr = nbr_idx.shape[2]
    assert (b, at, nbr, d) == (B, At, Nbr, D)
    nbr3 = nbr_idx.astype(jnp.int32).reshape(b, NBF // 128, 128)
    out_t = _node_k_t(node_embedding, nbr3)
    return jnp.transpose(out_t, (0, 4, 1, 2, 3))


# speedup vs baseline: 2.5514x; 2.5514x over previous
import functools

import jax
import jax.numpy as jnp
from jax import lax
from jax.experimental import pallas as pl
from jax.experimental.pallas import tpu as pltpu
from jax.experimental.pallas import tpu_sc as plsc

B, At, Nbr, D = 2, 512, 24, 64
NC, NS = 2, 16
SLABS = Nbr * Nbr
SPT = SLABS // NS
NBF = At * Nbr
LANES = 16
HALF = D // 2


def _mesh():
    return plsc.VectorSubcoreMesh(
        core_axis_name="c", subcore_axis_name="s", num_cores=NC, num_subcores=NS
    )


@functools.partial(
    pl.kernel,
    out_type=jax.ShapeDtypeStruct((B, Nbr, Nbr, D, At), jnp.float32),
    mesh=_mesh(),
    scratch_types=[
        pltpu.VMEM((D, At), jnp.float32),
        pltpu.VMEM((Nbr, At), jnp.int32),
        pltpu.VMEM((At,), jnp.int32),
        pltpu.VMEM((HALF, At), jnp.float32),
        pltpu.VMEM((HALF, At), jnp.float32),
        pltpu.SemaphoreType.DMA,
        pltpu.SemaphoreType.DMA,
    ],
    compiler_params=pltpu.CompilerParams(needs_layout_passes=False),
)
def _node_k_t(emb, nbr, out, emb_v, nbr_v, t2_ref, buf0, buf1, sem0, sem1):
    c = lax.axis_index("c")
    s = lax.axis_index("s")
    pltpu.sync_copy(emb.at[c], emb_v)
    pltpu.sync_copy(nbr.at[c], nbr_v)
    m0 = s * SPT

    def drain(buf, sem):
        pltpu.make_async_copy(out.at[0, 0, 0, pl.ds(0, HALF)], buf, sem).wait()

    def fill(buf, base_dd):
        def body(ablk, carry):
            t2v = t2_ref[pl.ds(ablk * LANES, LANES)]
            for dl in range(HALF):
                row = jnp.full((LANES,), base_dd + dl, jnp.int32)
                buf[dl, pl.ds(ablk * LANES, LANES)] = plsc.load_gather(
                    emb_v, [row, t2v]
                )
            return carry

        lax.fori_loop(0, At // LANES, body, 0)

    def step(g, carry):
        m = m0 + g
        j = m // Nbr
        k = m - j * Nbr

        def build(ablk, carry2):
            a = ablk * LANES + lax.iota(jnp.int32, LANES)
            jv = jnp.full((LANES,), 0, jnp.int32) + j
            kv = jnp.full((LANES,), 0, jnp.int32) + k
            t1 = plsc.load_gather(nbr_v, [jv, a])
            t2 = plsc.load_gather(nbr_v, [kv, t1])
            t2_ref[pl.ds(ablk * LANES, LANES)] = t2
            return carry2

        lax.fori_loop(0, At // LANES, build, 0)

        @pl.when(g > 0)
        def _():
            drain(buf0, sem0)

        fill(buf0, 0)
        pltpu.async_copy(buf0, out.at[c, j, k, pl.ds(0, HALF)], sem0)

        @pl.when(g > 0)
        def _():
            drain(buf1, sem1)

        fill(buf1, HALF)
        pltpu.async_copy(buf1, out.at[c, j, k, pl.ds(HALF, HALF)], sem1)
        return carry

    lax.fori_loop(0, SPT, step, 0)
    drain(buf0, sem0)
    drain(buf1, sem1)


def kernel(node_embedding, nbr_idx):
    b, at, d = node_embedding.shape
    nbr = nbr_idx.shape[2]
    assert (b, at, nbr, d) == (B, At, Nbr, D)
    emb_t = jnp.transpose(node_embedding, (0, 2, 1))
    nbr_t = jnp.transpose(nbr_idx.astype(jnp.int32), (0, 2, 1))
    out_t = _node_k_t(emb_t, nbr_t)
    return jnp.transpose(out_t, (0, 4, 1, 2, 3))


# speedup vs baseline: 5.0036x; 1.9611x over previous
import functools

import jax
import jax.numpy as jnp
from jax import lax
from jax.experimental import pallas as pl
from jax.experimental.pallas import tpu as pltpu
from jax.experimental.pallas import tpu_sc as plsc

B, At, Nbr, D = 2, 512, 24, 64
NC, NS = 2, 16
SLABS = Nbr * Nbr
SPT = SLABS // NS
NBF = At * Nbr
LANES = 16
HALF = D // 2


def _mesh():
    return plsc.VectorSubcoreMesh(
        core_axis_name="c", subcore_axis_name="s", num_cores=NC, num_subcores=NS
    )


@functools.partial(
    pl.kernel,
    out_type=jax.ShapeDtypeStruct((B, Nbr, Nbr, D, At), jnp.float32),
    mesh=_mesh(),
    scratch_types=[
        pltpu.VMEM((D, At), jnp.float32),
        pltpu.VMEM((Nbr, At), jnp.int32),
        pltpu.VMEM((At,), jnp.int32),
        pltpu.VMEM((HALF, At), jnp.float32),
        pltpu.VMEM((HALF, At), jnp.float32),
        pltpu.SemaphoreType.DMA,
        pltpu.SemaphoreType.DMA,
    ],
    compiler_params=pltpu.CompilerParams(needs_layout_passes=False),
)
def _node_k_t(emb, nbr, out, emb_v, nbr_v, t2_ref, buf0, buf1, sem0, sem1):
    c = lax.axis_index("c")
    s = lax.axis_index("s")
    pltpu.sync_copy(emb.at[c], emb_v)
    pltpu.sync_copy(nbr.at[c], nbr_v)
    m0 = s * SPT

    def drain(buf, sem):
        pltpu.make_async_copy(out.at[0, 0, 0, pl.ds(0, HALF)], buf, sem).wait()

    def fill(buf, base_dd):
        def body(ablk, carry):
            t2v = t2_ref[pl.ds(ablk * LANES, LANES)]
            for dl4 in range(0, HALF, 8):
                gs = []
                for i in range(8):
                    row = jnp.full((LANES,), base_dd + dl4 + i, jnp.int32)
                    gs.append(plsc.load_gather(emb_v, [row, t2v]))
                for i in range(8):
                    buf[dl4 + i, pl.ds(ablk * LANES, LANES)] = gs[i]
            return carry

        lax.fori_loop(0, At // LANES, body, 0)

    def step(g, carry):
        m = m0 + g
        j = m // Nbr
        k = m - j * Nbr

        def build(ablk, carry2):
            a = ablk * LANES + lax.iota(jnp.int32, LANES)
            jv = jnp.full((LANES,), 0, jnp.int32) + j
            kv = jnp.full((LANES,), 0, jnp.int32) + k
            t1 = plsc.load_gather(nbr_v, [jv, a])
            t2 = plsc.load_gather(nbr_v, [kv, t1])
            t2_ref[pl.ds(ablk * LANES, LANES)] = t2
            return carry2

        lax.fori_loop(0, At // LANES, build, 0)

        @pl.when(g > 0)
        def _():
            drain(buf0, sem0)

        fill(buf0, 0)
        pltpu.async_copy(buf0, out.at[c, j, k, pl.ds(0, HALF)], sem0)

        @pl.when(g > 0)
        def _():
            drain(buf1, sem1)

        fill(buf1, HALF)
        pltpu.async_copy(buf1, out.at[c, j, k, pl.ds(HALF, HALF)], sem1)
        return carry

    lax.fori_loop(0, SPT, step, 0)
    drain(buf0, sem0)
    drain(buf1, sem1)


def kernel(node_embedding, nbr_idx):
    b, at, d = node_embedding.shape
    nbr = nbr_idx.shape[2]
    assert (b, at, nbr, d) == (B, At, Nbr, D)
    emb_t = jnp.transpose(node_embedding, (0, 2, 1))
    nbr_t = jnp.transpose(nbr_idx.astype(jnp.int32), (0, 2, 1))
    out_t = _node_k_t(emb_t, nbr_t)
    return jnp.transpose(out_t, (0, 4, 1, 2, 3))
